# Initial kernel scaffold; baseline (speedup 1.0000x reference)
import jax
import jax.numpy as jnp
from jax.experimental import pallas as pl


def kernel(x, stn_conv1_w, stn_conv1_b, stn_bn1_g, stn_bn1_b, stn_conv2_w, stn_conv2_b, stn_bn2_g, stn_bn2_b, stn_fc1_w, stn_fc1_b, stn_bn3_g, stn_bn3_b, stn_fc2_w, stn_fc2_b, conv1_w, conv1_b, bn1_g, bn1_b, conv2_w, conv2_b, bn2_g, bn2_b):
    raise NotImplementedError("write your pallas kernel here")


# speedup vs baseline: 4.5917x; 4.5917x over previous
import functools

import jax
import jax.numpy as jnp
from jax.experimental import pallas as pl
from jax.experimental.pallas import tpu as pltpu

EPS = 1e-5
_VMEM_LIMIT = 64 * 1024 * 1024


def _pick_gb(b, target):
    if b <= target:
        return b
    for t in range(target, 0, -1):
        if b % t == 0:
            return t
    return 1


def _moments_kernel(n, x_ref, s0_ref, s1_ref, q00_ref, q11_ref, q01_ref):
    xb = x_ref[...]
    x0 = xb[:, :n]
    x1 = xb[:, n:]
    s0_ref[...] = jnp.sum(x0, axis=1, keepdims=True)
    s1_ref[...] = jnp.sum(x1, axis=1, keepdims=True)
    q00_ref[...] = jnp.sum(x0 * x0, axis=1, keepdims=True)
    q11_ref[...] = jnp.sum(x1 * x1, axis=1, keepdims=True)
    q01_ref[...] = jnp.sum(x0 * x1, axis=1, keepdims=True)


def _pair_kernel(gb, per_batch, x_ref, lhs_ref, sh_ref, w2t_ref,
                 sum_ref, ssq_ref, mx_ref, mn_ref):
    sh = sh_ref[...]
    w2t = w2t_ref[...]
    for g in range(gb):
        xg = x_ref[g]
        lg = lhs_ref[g] if per_batch else lhs_ref[0]
        y1 = jnp.dot(lg, xg, preferred_element_type=jnp.float32,
                     precision=jax.lax.Precision.HIGHEST)
        a = jnp.maximum(y1 + sh, 0.0)
        y2 = jnp.dot(w2t, a, preferred_element_type=jnp.float32)
        sum_ref[0, :, g:g + 1] = jnp.sum(y2, axis=1, keepdims=True)
        ssq_ref[0, :, g:g + 1] = jnp.sum(y2 * y2, axis=1, keepdims=True)
        mx_ref[0, :, g:g + 1] = jnp.max(y2, axis=1, keepdims=True)
        mn_ref[0, :, g:g + 1] = jnp.min(y2, axis=1, keepdims=True)


def _head_kernel(mx_ref, mn_ref, b2_ref, sc2_ref, sh2_ref, w3t_ref, b3_ref,
                 g3_ref, bt3_ref, w4t_ref, b4e_ref, o_ref):
    sc = sc2_ref[...]
    mxf = mx_ref[...] + b2_ref[...]
    mnf = mn_ref[...] + b2_ref[...]
    g = jnp.where(sc >= 0.0, sc * mxf, sc * mnf) + sh2_ref[...]
    g = jnp.maximum(g, 0.0)
    h = jnp.dot(w3t_ref[...], g, preferred_element_type=jnp.float32) + b3_ref[...]
    m = jnp.mean(h, axis=1, keepdims=True)
    v = jnp.maximum(jnp.mean(h * h, axis=1, keepdims=True) - m * m, 0.0)
    h = (h - m) * jax.lax.rsqrt(v + EPS) * g3_ref[...] + bt3_ref[...]
    h = jnp.maximum(h, 0.0)
    o_ref[...] = (jnp.dot(w4t_ref[...], h, preferred_element_type=jnp.float32)
                  + b4e_ref[...])


def _moments(x):
    B, d, N = x.shape
    gbm = _pick_gb(B, 128)
    xf = x.reshape(B, d * N)
    out = jax.ShapeDtypeStruct((B, 1), jnp.float32)
    ospec = pl.BlockSpec((gbm, 1), lambda i: (i, 0))
    return pl.pallas_call(
        functools.partial(_moments_kernel, N),
        out_shape=(out,) * 5,
        grid=(B // gbm,),
        in_specs=[pl.BlockSpec((gbm, d * N), lambda i: (i, 0))],
        out_specs=(ospec,) * 5,
        compiler_params=pltpu.CompilerParams(
            dimension_semantics=("parallel",),
            vmem_limit_bytes=_VMEM_LIMIT),
    )(xf)


def _pair(x, lhs, sh, w2t):
    B, d, N = x.shape
    Bl, C1, _ = lhs.shape
    C2 = w2t.shape[0]
    gb = _pick_gb(B, 32)
    per_batch = Bl == B
    lhs_spec = (pl.BlockSpec((gb, C1, d), lambda i: (i, 0, 0)) if per_batch
                else pl.BlockSpec((1, C1, d), lambda i: (0, 0, 0)))
    out = jax.ShapeDtypeStruct((B // gb, C2, gb), jnp.float32)
    ospec = pl.BlockSpec((1, C2, gb), lambda i: (i, 0, 0))
    res = pl.pallas_call(
        functools.partial(_pair_kernel, gb, per_batch),
        out_shape=(out,) * 4,
        grid=(B // gb,),
        in_specs=[
            pl.BlockSpec((gb, d, N), lambda i: (i, 0, 0)),
            lhs_spec,
            pl.BlockSpec((C1, 1), lambda i: (0, 0)),
            pl.BlockSpec((C2, C1), lambda i: (0, 0)),
        ],
        out_specs=(ospec,) * 4,
        compiler_params=pltpu.CompilerParams(
            dimension_semantics=("parallel",),
            vmem_limit_bytes=_VMEM_LIMIT),
    )(x, lhs, sh, w2t)
    return tuple(r.transpose(1, 0, 2).reshape(C2, B) for r in res)


def _head(mx, mn, b2c, sc2c, sh2c, w3t, b3c, g3c, bt3c, w4t, b4ec):
    C2, B = mx.shape
    H = w3t.shape[0]
    dd = w4t.shape[0]

    def full(shape):
        return pl.BlockSpec(shape, lambda i, _s=shape: tuple(0 for _ in _s))

    return pl.pallas_call(
        _head_kernel,
        out_shape=jax.ShapeDtypeStruct((dd, B), jnp.float32),
        grid=(1,),
        in_specs=[
            full((C2, B)), full((C2, B)), full((C2, 1)), full((C2, 1)),
            full((C2, 1)), full((H, C2)), full((H, 1)), full((H, 1)),
            full((H, 1)), full((dd, H)), full((dd, 1)),
        ],
        out_specs=full((dd, B)),
        compiler_params=pltpu.CompilerParams(
            dimension_semantics=("arbitrary",),
            vmem_limit_bytes=_VMEM_LIMIT),
    )(mx, mn, b2c, sc2c, sh2c, w3t, b3c, g3c, bt3c, w4t, b4ec)


def _fold_from_totals(total_sum, total_ssq, n_rows, gamma, beta):
    mean = total_sum / n_rows
    var = jnp.maximum(total_ssq / n_rows - mean * mean, 0.0)
    scale = gamma * jax.lax.rsqrt(var + EPS)
    shift = beta - mean * scale
    return scale, shift


def _fold_conv2(sum_t, ssq_t, b2, n_rows, gamma, beta):
    s = jnp.sum(sum_t, axis=1)
    q = jnp.sum(ssq_t, axis=1)
    total_sum = s + n_rows * b2
    total_ssq = q + 2.0 * b2 * s + n_rows * b2 * b2
    return _fold_from_totals(total_sum, total_ssq, n_rows, gamma, beta)


def kernel(x, stn_conv1_w, stn_conv1_b, stn_bn1_g, stn_bn1_b, stn_conv2_w,
           stn_conv2_b, stn_bn2_g, stn_bn2_b, stn_fc1_w, stn_fc1_b,
           stn_bn3_g, stn_bn3_b, stn_fc2_w, stn_fc2_b, conv1_w, conv1_b,
           bn1_g, bn1_b, conv2_w, conv2_b, bn2_g, bn2_b):
    B, d, N = x.shape
    x = x.astype(jnp.float32)
    C1 = stn_conv1_w.shape[1]
    C2 = stn_conv2_w.shape[1]
    Ct1 = conv1_w.shape[1]
    Ct2 = conv2_w.shape[1]
    nr = float(B * N)
    fN = float(N)

    s0, s1, q00, q11, q01 = _moments(x)

    S0, S1 = jnp.sum(s0), jnp.sum(s1)
    Q00, Q11, Q01 = jnp.sum(q00), jnp.sum(q11), jnp.sum(q01)
    w0, w1 = stn_conv1_w[0], stn_conv1_w[1]
    b1 = stn_conv1_b
    tsum = S0 * w0 + S1 * w1 + nr * b1
    tssq = (w0 * w0 * Q00 + w1 * w1 * Q11 + 2.0 * w0 * w1 * Q01
            + 2.0 * b1 * (w0 * S0 + w1 * S1) + nr * b1 * b1)
    sc1, sh1 = _fold_from_totals(tsum, tssq, nr, stn_bn1_g, stn_bn1_b)
    lhs1 = (sc1[:, None] * stn_conv1_w.T)[None]
    sh1c = (sc1 * stn_conv1_b + sh1).reshape(C1, 1)
    w2t = stn_conv2_w.T

    sum2, ssq2, mx2, mn2 = _pair(x, lhs1, sh1c, w2t)
    sc2, sh2 = _fold_conv2(sum2, ssq2, stn_conv2_b, nr, stn_bn2_g, stn_bn2_b)

    eye = jnp.eye(d, dtype=jnp.float32).reshape(d * d)
    trans_t = _head(
        mx2, mn2, stn_conv2_b.reshape(C2, 1), sc2.reshape(C2, 1),
        sh2.reshape(C2, 1), stn_fc1_w.T, stn_fc1_b.reshape(-1, 1),
        stn_bn3_g.reshape(-1, 1), stn_bn3_b.reshape(-1, 1), stn_fc2_w.T,
        (stn_fc2_b + eye).reshape(d * d, 1))
    trans = trans_t.T.reshape(B, d, d)

    eff = jnp.einsum("bij,jc->bic", trans, conv1_w)
    e0, e1 = eff[:, 0, :], eff[:, 1, :]
    cb = conv1_b
    psum = e0 * s0 + e1 * s1 + fN * cb
    pssq = (e0 * e0 * q00 + e1 * e1 * q11 + 2.0 * e0 * e1 * q01
            + 2.0 * cb * (e0 * s0 + e1 * s1) + fN * cb * cb)
    tsc1, tsh1 = _fold_from_totals(jnp.sum(psum, axis=0), jnp.sum(pssq, axis=0),
                                   nr, bn1_g, bn1_b)
    lhs_t = (eff * tsc1[None, None, :]).transpose(0, 2, 1)
    sht_c = (tsc1 * conv1_b + tsh1).reshape(Ct1, 1)
    w2t_t = conv2_w.T

    sumt, ssqt, mxt, mnt = _pair(x, lhs_t, sht_c, w2t_t)
    tsc2, tsh2 = _fold_conv2(sumt, ssqt, conv2_b, nr, bn2_g, bn2_b)

    scc = tsc2.reshape(Ct2, 1)
    cb2 = conv2_b.reshape(Ct2, 1)
    gfeat_t = jnp.where(scc >= 0.0, scc * (mxt + cb2), scc * (mnt + cb2)) \
        + tsh2.reshape(Ct2, 1)
    return gfeat_t.T, trans, None


# speedup vs baseline: 9.3676x; 2.0401x over previous
import functools

import jax
import jax.numpy as jnp
from jax.experimental import pallas as pl
from jax.experimental.pallas import tpu as pltpu

EPS = 1e-5
_VMEM_LIMIT = 64 * 1024 * 1024


def _pick_gb(b, target):
    if b <= target:
        return b
    for t in range(target, 0, -1):
        if b % t == 0:
            return t
    return 1


def _moments_kernel(n, x_ref, s0_ref, s1_ref, q00_ref, q11_ref, q01_ref):
    xb = x_ref[...]
    x0 = xb[:, :n]
    x1 = xb[:, n:]
    s0_ref[...] = jnp.sum(x0, axis=1, keepdims=True)
    s1_ref[...] = jnp.sum(x1, axis=1, keepdims=True)
    q00_ref[...] = jnp.sum(x0 * x0, axis=1, keepdims=True)
    q11_ref[...] = jnp.sum(x1 * x1, axis=1, keepdims=True)
    q01_ref[...] = jnp.sum(x0 * x1, axis=1, keepdims=True)


def _pair_kernel(gb, per_batch, x_ref, lhs_ref, sh_ref, w2t_ref,
                 sum_ref, ssq_ref, mx_ref, mn_ref):
    sh = sh_ref[...]
    w2t = w2t_ref[...]
    for g in range(gb):
        xg = x_ref[g]
        lg = lhs_ref[g] if per_batch else lhs_ref[0]
        y1 = (lg[:, 0:1] * xg[0:1, :] + lg[:, 1:2] * xg[1:2, :] + sh)
        a = jnp.maximum(y1, 0.0)
        y2 = jnp.dot(w2t, a, preferred_element_type=jnp.float32)
        sum_ref[0, :, g:g + 1] = jnp.sum(y2, axis=1, keepdims=True)
        ssq_ref[0, :, g:g + 1] = jnp.sum(y2 * y2, axis=1, keepdims=True)
        mx_ref[0, :, g:g + 1] = jnp.max(y2, axis=1, keepdims=True)
        mn_ref[0, :, g:g + 1] = jnp.min(y2, axis=1, keepdims=True)


def _head_kernel(mx_ref, mn_ref, b2_ref, sc2_ref, sh2_ref, w3t_ref, b3_ref,
                 g3_ref, bt3_ref, w4t_ref, b4e_ref, o_ref):
    sc = sc2_ref[...]
    mxf = mx_ref[...] + b2_ref[...]
    mnf = mn_ref[...] + b2_ref[...]
    g = jnp.where(sc >= 0.0, sc * mxf, sc * mnf) + sh2_ref[...]
    g = jnp.maximum(g, 0.0)
    h = jnp.dot(w3t_ref[...], g, preferred_element_type=jnp.float32) + b3_ref[...]
    m = jnp.mean(h, axis=1, keepdims=True)
    v = jnp.maximum(jnp.mean(h * h, axis=1, keepdims=True) - m * m, 0.0)
    h = (h - m) * jax.lax.rsqrt(v + EPS) * g3_ref[...] + bt3_ref[...]
    h = jnp.maximum(h, 0.0)
    o_ref[...] = (jnp.dot(w4t_ref[...], h, preferred_element_type=jnp.float32)
                  + b4e_ref[...])


def _moments(x):
    B, d, N = x.shape
    gbm = _pick_gb(B, 128)
    xf = x.reshape(B, d * N)
    out = jax.ShapeDtypeStruct((B, 1), jnp.float32)
    ospec = pl.BlockSpec((gbm, 1), lambda i: (i, 0))
    return pl.pallas_call(
        functools.partial(_moments_kernel, N),
        out_shape=(out,) * 5,
        grid=(B // gbm,),
        in_specs=[pl.BlockSpec((gbm, d * N), lambda i: (i, 0))],
        out_specs=(ospec,) * 5,
        compiler_params=pltpu.CompilerParams(
            dimension_semantics=("parallel",),
            vmem_limit_bytes=_VMEM_LIMIT),
    )(xf)


def _pair(x, lhs, sh, w2t):
    B, d, N = x.shape
    Bl, C1, _ = lhs.shape
    C2 = w2t.shape[0]
    gb = _pick_gb(B, 32)
    per_batch = Bl == B
    lhs_spec = (pl.BlockSpec((gb, C1, d), lambda i: (i, 0, 0)) if per_batch
                else pl.BlockSpec((1, C1, d), lambda i: (0, 0, 0)))
    out = jax.ShapeDtypeStruct((B // gb, C2, gb), jnp.float32)
    ospec = pl.BlockSpec((1, C2, gb), lambda i: (i, 0, 0))
    res = pl.pallas_call(
        functools.partial(_pair_kernel, gb, per_batch),
        out_shape=(out,) * 4,
        grid=(B // gb,),
        in_specs=[
            pl.BlockSpec((gb, d, N), lambda i: (i, 0, 0)),
            lhs_spec,
            pl.BlockSpec((C1, 1), lambda i: (0, 0)),
            pl.BlockSpec((C2, C1), lambda i: (0, 0)),
        ],
        out_specs=(ospec,) * 4,
        compiler_params=pltpu.CompilerParams(
            dimension_semantics=("parallel",),
            vmem_limit_bytes=_VMEM_LIMIT),
    )(x, lhs, sh, w2t)
    return tuple(r.transpose(1, 0, 2).reshape(C2, B) for r in res)


def _head(mx, mn, b2c, sc2c, sh2c, w3t, b3c, g3c, bt3c, w4t, b4ec):
    C2, B = mx.shape
    H = w3t.shape[0]
    dd = w4t.shape[0]

    def full(shape):
        return pl.BlockSpec(shape, lambda i, _s=shape: tuple(0 for _ in _s))

    return pl.pallas_call(
        _head_kernel,
        out_shape=jax.ShapeDtypeStruct((dd, B), jnp.float32),
        grid=(1,),
        in_specs=[
            full((C2, B)), full((C2, B)), full((C2, 1)), full((C2, 1)),
            full((C2, 1)), full((H, C2)), full((H, 1)), full((H, 1)),
            full((H, 1)), full((dd, H)), full((dd, 1)),
        ],
        out_specs=full((dd, B)),
        compiler_params=pltpu.CompilerParams(
            dimension_semantics=("arbitrary",),
            vmem_limit_bytes=_VMEM_LIMIT),
    )(mx, mn, b2c, sc2c, sh2c, w3t, b3c, g3c, bt3c, w4t, b4ec)


def _fold_from_totals(total_sum, total_ssq, n_rows, gamma, beta):
    mean = total_sum / n_rows
    var = jnp.maximum(total_ssq / n_rows - mean * mean, 0.0)
    scale = gamma * jax.lax.rsqrt(var + EPS)
    shift = beta - mean * scale
    return scale, shift


def _fold_conv2(sum_t, ssq_t, b2, n_rows, gamma, beta):
    s = jnp.sum(sum_t, axis=1)
    q = jnp.sum(ssq_t, axis=1)
    total_sum = s + n_rows * b2
    total_ssq = q + 2.0 * b2 * s + n_rows * b2 * b2
    return _fold_from_totals(total_sum, total_ssq, n_rows, gamma, beta)


def kernel(x, stn_conv1_w, stn_conv1_b, stn_bn1_g, stn_bn1_b, stn_conv2_w,
           stn_conv2_b, stn_bn2_g, stn_bn2_b, stn_fc1_w, stn_fc1_b,
           stn_bn3_g, stn_bn3_b, stn_fc2_w, stn_fc2_b, conv1_w, conv1_b,
           bn1_g, bn1_b, conv2_w, conv2_b, bn2_g, bn2_b):
    B, d, N = x.shape
    x = x.astype(jnp.float32)
    C1 = stn_conv1_w.shape[1]
    C2 = stn_conv2_w.shape[1]
    Ct1 = conv1_w.shape[1]
    Ct2 = conv2_w.shape[1]
    nr = float(B * N)
    fN = float(N)

    s0, s1, q00, q11, q01 = _moments(x)

    S0, S1 = jnp.sum(s0), jnp.sum(s1)
    Q00, Q11, Q01 = jnp.sum(q00), jnp.sum(q11), jnp.sum(q01)
    w0, w1 = stn_conv1_w[0], stn_conv1_w[1]
    b1 = stn_conv1_b
    tsum = S0 * w0 + S1 * w1 + nr * b1
    tssq = (w0 * w0 * Q00 + w1 * w1 * Q11 + 2.0 * w0 * w1 * Q01
            + 2.0 * b1 * (w0 * S0 + w1 * S1) + nr * b1 * b1)
    sc1, sh1 = _fold_from_totals(tsum, tssq, nr, stn_bn1_g, stn_bn1_b)
    lhs1 = (sc1[:, None] * stn_conv1_w.T)[None]
    sh1c = (sc1 * stn_conv1_b + sh1).reshape(C1, 1)
    w2t = stn_conv2_w.T

    sum2, ssq2, mx2, mn2 = _pair(x, lhs1, sh1c, w2t)
    sc2, sh2 = _fold_conv2(sum2, ssq2, stn_conv2_b, nr, stn_bn2_g, stn_bn2_b)

    eye = jnp.eye(d, dtype=jnp.float32).reshape(d * d)
    trans_t = _head(
        mx2, mn2, stn_conv2_b.reshape(C2, 1), sc2.reshape(C2, 1),
        sh2.reshape(C2, 1), stn_fc1_w.T, stn_fc1_b.reshape(-1, 1),
        stn_bn3_g.reshape(-1, 1), stn_bn3_b.reshape(-1, 1), stn_fc2_w.T,
        (stn_fc2_b + eye).reshape(d * d, 1))
    trans = trans_t.T.reshape(B, d, d)

    eff = jnp.einsum("bij,jc->bic", trans, conv1_w)
    e0, e1 = eff[:, 0, :], eff[:, 1, :]
    cb = conv1_b
    psum = e0 * s0 + e1 * s1 + fN * cb
    pssq = (e0 * e0 * q00 + e1 * e1 * q11 + 2.0 * e0 * e1 * q01
            + 2.0 * cb * (e0 * s0 + e1 * s1) + fN * cb * cb)
    tsc1, tsh1 = _fold_from_totals(jnp.sum(psum, axis=0), jnp.sum(pssq, axis=0),
                                   nr, bn1_g, bn1_b)
    lhs_t = (eff * tsc1[None, None, :]).transpose(0, 2, 1)
    sht_c = (tsc1 * conv1_b + tsh1).reshape(Ct1, 1)
    w2t_t = conv2_w.T

    sumt, ssqt, mxt, mnt = _pair(x, lhs_t, sht_c, w2t_t)
    tsc2, tsh2 = _fold_conv2(sumt, ssqt, conv2_b, nr, bn2_g, bn2_b)

    scc = tsc2.reshape(Ct2, 1)
    cb2 = conv2_b.reshape(Ct2, 1)
    gfeat_t = jnp.where(scc >= 0.0, scc * (mxt + cb2), scc * (mnt + cb2)) \
        + tsh2.reshape(Ct2, 1)
    return gfeat_t.T, trans, None
